# baseline (speedup 1.0000x reference)
import functools

import jax
import jax.numpy as jnp
from jax import lax
from jax.experimental import pallas as pl
from jax.experimental.pallas import tpu as pltpu
from jax.experimental.pallas import tpu_sc as plsc

C = 19
L = 16
NC, NS = 2, 16
NW = NC * NS
B = 8
H = 512
W = 512
ROWS = 8
COLS = 256
TB = 2
SCB = B - TB
CHUNK_PX = ROWS * COLS
CHUNKS_PER_IMG = (H // ROWS) * (W // COLS)
NCHUNKS = SCB * CHUNKS_PER_IMG
CHUNKS_PER_TILE = NCHUNKS // NW
UNROLL = 2


def _argmax_tree(vals):
    n = len(vals)
    nodes = []
    for a in range(0, n - 1, 2):
        gt = vals[a + 1] > vals[a]
        v = jnp.where(gt, vals[a + 1], vals[a])
        i = jnp.where(gt, jnp.full((L,), a + 1, jnp.int32),
                      jnp.full((L,), a, jnp.int32))
        nodes.append((v, i))
    if n % 2:
        nodes.append((vals[n - 1], jnp.full((L,), n - 1, jnp.int32)))
    while len(nodes) > 1:
        nxt = []
        for k in range(0, len(nodes) - 1, 2):
            (va, ia), (vb, ib) = nodes[k], nodes[k + 1]
            gt = vb > va
            nxt.append((jnp.where(gt, vb, va), jnp.where(gt, ib, ia)))
        if len(nodes) % 2:
            nxt.append(nodes[-1])
        nodes = nxt
    return nodes[0][1]


@functools.partial(
    pl.kernel,
    out_type=(
        jax.ShapeDtypeStruct((NW, C, L), jnp.int32),
        jax.ShapeDtypeStruct((NW, C, L), jnp.int32),
        jax.ShapeDtypeStruct((NW, C, L), jnp.int32),
    ),
    mesh=plsc.VectorSubcoreMesh(core_axis_name="c", subcore_axis_name="s"),
    scratch_types=[
        pltpu.VMEM((2, C, ROWS, COLS), jnp.float32),
        pltpu.VMEM((2, ROWS, COLS), jnp.int32),
        pltpu.VMEM((C, L), jnp.int32),
        pltpu.VMEM((C, L), jnp.int32),
        pltpu.VMEM((C, L), jnp.int32),
        pltpu.SemaphoreType.DMA,
        pltpu.SemaphoreType.DMA,
    ],
    compiler_params=pltpu.CompilerParams(needs_layout_passes=False),
)
def _count_kernel(logits_hbm, targets_hbm, out_lo, out_hi, out_tg,
                  logit_v, tgt_v, hist_lo, hist_hi, hist_tg, sem0, sem1):
    wid = lax.axis_index("s") * NC + lax.axis_index("c")
    q_base = wid * CHUNKS_PER_TILE

    zeros = jnp.zeros((L,), jnp.int32)
    for r in range(C):
        hist_lo[r, :] = zeros
        hist_hi[r, :] = zeros
        hist_tg[r, :] = zeros

    lane = lax.iota(jnp.int32, L)
    ones = jnp.ones((L,), jnp.int32)
    sems = (sem0, sem1)

    def copies(q, buf, sem):
        b = q // CHUNKS_PER_IMG
        rem = q % CHUNKS_PER_IMG
        r0 = (rem // 2) * ROWS
        c0 = (rem % 2) * COLS
        return (
            pltpu.make_async_copy(
                logits_hbm.at[b, :, pl.ds(r0, ROWS), pl.ds(c0, COLS)],
                logit_v.at[buf], sem),
            pltpu.make_async_copy(
                targets_hbm.at[b, pl.ds(r0, ROWS), pl.ds(c0, COLS)],
                tgt_v.at[buf], sem),
        )

    def start(q, buf, sem):
        for cp in copies(q, buf, sem):
            cp.start()

    def wait(q, buf, sem):
        for cp in copies(q, buf, sem):
            cp.wait()

    start(q_base, 0, sem0)
    start(q_base + 1, 1, sem1)

    def compute_chunk(buf):
        @plsc.parallel_loop(0, ROWS, 1)
        def row_body(r):
            @plsc.parallel_loop(0, COLS // L, 1, unroll=UNROLL)
            def vec_body(j):
                off = j * L
                vals = [logit_v[buf, c, r, pl.ds(off, L)]
                        for c in range(C)]
                pred = _argmax_tree(vals)
                t = tgt_v[buf, r, pl.ds(off, L)]
                eqm = pred == t
                plsc.addupdate_scatter(hist_hi, [pred, lane], ones,
                                       mask=eqm)
                plsc.addupdate_scatter(hist_lo, [pred, lane], ones,
                                       mask=jnp.logical_not(eqm))
                plsc.addupdate_scatter(hist_tg, [t, lane], ones)

    def pair_body(i, carry):
        base = q_base + 2 * i
        for p in range(2):
            q = base + p
            wait(q, p, sems[p])
            compute_chunk(p)

            @pl.when(2 * i + p + 2 < CHUNKS_PER_TILE)
            def _():
                start(q + 2, p, sems[p])
        return carry

    lax.fori_loop(0, CHUNKS_PER_TILE // 2, pair_body, 0)

    pltpu.sync_copy(hist_lo, out_lo.at[wid])
    pltpu.sync_copy(hist_hi, out_hi.at[wid])
    pltpu.sync_copy(hist_tg, out_tg.at[wid])


def _tc_count_body(x_ref, t_ref, o_ref, acc_ref):
    bi = pl.program_id(0)
    rj = pl.program_id(1)

    @pl.when((bi == 0) & (rj == 0))
    def _init():
        acc_ref[...] = jnp.zeros_like(acc_ref)

    x = x_ref[0]
    t = t_ref[0]
    m = x[0]
    for c in range(1, C):
        m = jnp.maximum(m, x[c])
    pred = jnp.full(t.shape, C - 1, jnp.int32)
    for c in range(C - 2, -1, -1):
        pred = jnp.where(x[c] == m, c, pred)
    for c in range(C):
        pm = pred == c
        tm = t == c
        hit = pm & tm
        acc_ref[0, c] += (pm & jnp.logical_not(tm)).astype(jnp.float32)
        acc_ref[1, c] += hit.astype(jnp.float32)
        acc_ref[2, c] += tm.astype(jnp.float32)

    @pl.when((bi == TB - 1) & (rj == (H // ROWS) - 1))
    def _fin():
        o_ref[...] = jnp.sum(acc_ref[...], axis=(2, 3))


_tc_count = pl.pallas_call(
    _tc_count_body,
    grid=(TB, H // ROWS),
    in_specs=[
        pl.BlockSpec((1, C, ROWS, W), lambda i, j: (SCB + i, 0, j, 0)),
        pl.BlockSpec((1, ROWS, W), lambda i, j: (SCB + i, j, 0)),
    ],
    out_specs=pl.BlockSpec((3, C), lambda i, j: (0, 0)),
    out_shape=jax.ShapeDtypeStruct((3, C), jnp.float32),
    scratch_shapes=[pltpu.VMEM((3, C, ROWS, W), jnp.float32)],
)


def _combine_body(lo_ref, hi_ref, tg_ref, tc_ref, out_ref):
    lo = jnp.sum(lo_ref[...].astype(jnp.float32), axis=(0, 2)) + tc_ref[0]
    hi = jnp.sum(hi_ref[...].astype(jnp.float32), axis=(0, 2)) + tc_ref[1]
    tg = jnp.sum(tg_ref[...].astype(jnp.float32), axis=(0, 2)) + tc_ref[2]
    inter = hi
    pred = lo + hi
    union = pred + tg - inter
    iou = inter / (union + 1e-16)
    out_ref[0, 0] = jnp.sum(iou) / float(C)


_combine = pl.pallas_call(
    _combine_body,
    out_shape=jax.ShapeDtypeStruct((1, 1), jnp.float32),
    out_specs=pl.BlockSpec(memory_space=pltpu.SMEM),
)


def kernel(inputs, targets):
    lo, hi, tg = _count_kernel(inputs, targets)
    tc = _tc_count(inputs, targets)
    return _combine(lo, hi, tg, tc)[0, 0]


# speedup vs baseline: 1.2675x; 1.2675x over previous
import functools

import jax
import jax.numpy as jnp
from jax import lax
from jax.experimental import pallas as pl
from jax.experimental.pallas import tpu as pltpu
from jax.experimental.pallas import tpu_sc as plsc

C = 19
L = 16
NC, NS = 2, 16
NW = NC * NS
B = 8
H = 512
W = 512
ROWS = 8
COLS = 256
TB = 2
SCB = B - TB
CHUNK_PX = ROWS * COLS
CHUNKS_PER_IMG = (H // ROWS) * (W // COLS)
NCHUNKS = SCB * CHUNKS_PER_IMG
CHUNKS_PER_TILE = NCHUNKS // NW
UNROLL = 2


def _argmax_tree(vals):
    n = len(vals)
    nodes = []
    for a in range(0, n - 1, 2):
        gt = vals[a + 1] > vals[a]
        v = jnp.where(gt, vals[a + 1], vals[a])
        i = jnp.where(gt, jnp.full((L,), a + 1, jnp.int32),
                      jnp.full((L,), a, jnp.int32))
        nodes.append((v, i))
    if n % 2:
        nodes.append((vals[n - 1], jnp.full((L,), n - 1, jnp.int32)))
    while len(nodes) > 1:
        nxt = []
        for k in range(0, len(nodes) - 1, 2):
            (va, ia), (vb, ib) = nodes[k], nodes[k + 1]
            gt = vb > va
            nxt.append((jnp.where(gt, vb, va), jnp.where(gt, ib, ia)))
        if len(nodes) % 2:
            nxt.append(nodes[-1])
        nodes = nxt
    return nodes[0][1]


@functools.partial(
    pl.kernel,
    out_type=(
        jax.ShapeDtypeStruct((NW, C, L), jnp.int32),
        jax.ShapeDtypeStruct((NW, C, L), jnp.int32),
        jax.ShapeDtypeStruct((NW, C, L), jnp.int32),
    ),
    mesh=plsc.VectorSubcoreMesh(core_axis_name="c", subcore_axis_name="s"),
    scratch_types=[
        pltpu.VMEM((2, C, ROWS, COLS), jnp.float32),
        pltpu.VMEM((2, ROWS, COLS), jnp.int32),
        pltpu.VMEM((C, L), jnp.int32),
        pltpu.VMEM((C, L), jnp.int32),
        pltpu.VMEM((C, L), jnp.int32),
        pltpu.SemaphoreType.DMA,
        pltpu.SemaphoreType.DMA,
    ],
    compiler_params=pltpu.CompilerParams(needs_layout_passes=False),
)
def _count_kernel(logits_hbm, targets_hbm, out_lo, out_hi, out_tg,
                  logit_v, tgt_v, hist_lo, hist_hi, hist_tg, sem0, sem1):
    wid = lax.axis_index("s") * NC + lax.axis_index("c")
    q_base = wid * CHUNKS_PER_TILE

    zeros = jnp.zeros((L,), jnp.int32)
    for r in range(C):
        hist_lo[r, :] = zeros
        hist_hi[r, :] = zeros
        hist_tg[r, :] = zeros

    lane = lax.iota(jnp.int32, L)
    ones = jnp.ones((L,), jnp.int32)
    sems = (sem0, sem1)

    def copies(q, buf, sem):
        b = q // CHUNKS_PER_IMG
        rem = q % CHUNKS_PER_IMG
        r0 = (rem // 2) * ROWS
        c0 = (rem % 2) * COLS
        return (
            pltpu.make_async_copy(
                logits_hbm.at[b, :, pl.ds(r0, ROWS), pl.ds(c0, COLS)],
                logit_v.at[buf], sem),
            pltpu.make_async_copy(
                targets_hbm.at[b, pl.ds(r0, ROWS), pl.ds(c0, COLS)],
                tgt_v.at[buf], sem),
        )

    def start(q, buf, sem):
        for cp in copies(q, buf, sem):
            cp.start()

    def wait(q, buf, sem):
        for cp in copies(q, buf, sem):
            cp.wait()

    start(q_base, 0, sem0)
    start(q_base + 1, 1, sem1)

    def compute_chunk(buf):
        @plsc.parallel_loop(0, ROWS, 1)
        def row_body(r):
            @plsc.parallel_loop(0, COLS // L, 1, unroll=UNROLL)
            def vec_body(j):
                off = j * L
                vals = [logit_v[buf, c, r, pl.ds(off, L)]
                        for c in range(C)]
                pred = _argmax_tree(vals)
                t = tgt_v[buf, r, pl.ds(off, L)]
                eqm = pred == t
                plsc.addupdate_scatter(hist_hi, [pred, lane], ones,
                                       mask=eqm)
                plsc.addupdate_scatter(hist_lo, [pred, lane], ones,
                                       mask=jnp.logical_not(eqm))
                plsc.addupdate_scatter(hist_tg, [t, lane], ones)

    def pair_body(i, carry):
        base = q_base + 2 * i
        for p in range(2):
            q = base + p
            wait(q, p, sems[p])
            compute_chunk(p)

            @pl.when(2 * i + p + 2 < CHUNKS_PER_TILE)
            def _():
                start(q + 2, p, sems[p])
        return carry

    lax.fori_loop(0, CHUNKS_PER_TILE // 2, pair_body, 0)

    pltpu.sync_copy(hist_lo, out_lo.at[wid])
    pltpu.sync_copy(hist_hi, out_hi.at[wid])
    pltpu.sync_copy(hist_tg, out_tg.at[wid])


TC_ROWS = 16


def _fold(mf):
    return ((mf[:, 0:128] + mf[:, 128:256])
            + (mf[:, 256:384] + mf[:, 384:512]))


def _tc_count_body(x_ref, t_ref, o_ref, acc_ref):
    bi = pl.program_id(0)
    rj = pl.program_id(1)

    @pl.when((bi == 0) & (rj == 0))
    def _init():
        acc_ref[...] = jnp.zeros_like(acc_ref)

    x = x_ref[0]
    t = t_ref[0]
    m = x[0]
    for c in range(1, C):
        m = jnp.maximum(m, x[c])
    pred = jnp.full(t.shape, C - 1, jnp.int32)
    for c in range(C - 2, -1, -1):
        pred = jnp.where(x[c] == m, c, pred)
    for c in range(C):
        pmf = (pred == c).astype(jnp.float32)
        tmf = (t == c).astype(jnp.float32)
        acc_ref[0, c] += _fold(pmf)
        acc_ref[1, c] += _fold(pmf * tmf)
        acc_ref[2, c] += _fold(tmf)

    @pl.when((bi == TB - 1) & (rj == (H // TC_ROWS) - 1))
    def _fin():
        s = jnp.sum(acc_ref[...], axis=(2, 3))
        o_ref[0, :] = s[0, :] - s[1, :]
        o_ref[1, :] = s[1, :]
        o_ref[2, :] = s[2, :]


_tc_count = pl.pallas_call(
    _tc_count_body,
    grid=(TB, H // TC_ROWS),
    in_specs=[
        pl.BlockSpec((1, C, TC_ROWS, W), lambda i, j: (SCB + i, 0, j, 0)),
        pl.BlockSpec((1, TC_ROWS, W), lambda i, j: (SCB + i, j, 0)),
    ],
    out_specs=pl.BlockSpec((3, C), lambda i, j: (0, 0)),
    out_shape=jax.ShapeDtypeStruct((3, C), jnp.float32),
    scratch_shapes=[pltpu.VMEM((3, C, TC_ROWS, 128), jnp.float32)],
)


def _combine_body(lo_ref, hi_ref, tg_ref, tc_ref, out_ref):
    lo = jnp.sum(lo_ref[...].astype(jnp.float32), axis=(0, 2)) + tc_ref[0]
    hi = jnp.sum(hi_ref[...].astype(jnp.float32), axis=(0, 2)) + tc_ref[1]
    tg = jnp.sum(tg_ref[...].astype(jnp.float32), axis=(0, 2)) + tc_ref[2]
    inter = hi
    pred = lo + hi
    union = pred + tg - inter
    iou = inter / (union + 1e-16)
    out_ref[0, 0] = jnp.sum(iou) / float(C)


_combine = pl.pallas_call(
    _combine_body,
    out_shape=jax.ShapeDtypeStruct((1, 1), jnp.float32),
    out_specs=pl.BlockSpec(memory_space=pltpu.SMEM),
)


def kernel(inputs, targets):
    lo, hi, tg = _count_kernel(inputs, targets)
    tc = _tc_count(inputs, targets)
    return _combine(lo, hi, tg, tc)[0, 0]
